# baseline (speedup 1.0000x reference)
import functools

import jax
import jax.numpy as jnp
from jax import lax
from jax.experimental import pallas as pl
from jax.experimental.pallas import tpu as pltpu
from jax.experimental.pallas import tpu_sc as plsc

B = 4096
L = 50
NP = B // 2
LP = 104
D = 300
DP = 304
V = 517015
VP = 517016
NW = 32
SPW = B // NW
PPW = NP // NW
NT = DP // 16
INV_L = 1.0 / L

_mesh = plsc.VectorSubcoreMesh(core_axis_name="c", subcore_axis_name="s")


def _reduce_pair(rows, out_v, w0):
    for half in (0, 1):
        def word_body(j, accs):
            return tuple(accs[t] + rows[j, pl.ds(t * 16, 16)]
                         for t in range(NT))

        init = tuple(jnp.zeros((16,), jnp.float32) for _ in range(NT))
        accs = lax.fori_loop(half * L, (half + 1) * L, word_body, init,
                             unroll=5)
        for t in range(NT):
            out_v[w0 + half, pl.ds(t * 16, 16)] = accs[t] * INV_L


@functools.partial(
    pl.kernel,
    out_type=jax.ShapeDtypeStruct((B, DP), jnp.float32),
    mesh=_mesh,
    scratch_types=[
        pltpu.VMEM((PPW, LP), jnp.int32),
        pltpu.VMEM((LP, DP), jnp.float32),
        pltpu.VMEM((LP, DP), jnp.float32),
        pltpu.VMEM((SPW, DP), jnp.float32),
        pltpu.SemaphoreType.DMA,
        pltpu.SemaphoreType.DMA,
    ],
    compiler_params=pltpu.CompilerParams(use_tc_tiling_on_sc=False),
)
def _pooled_lookup(idx_hbm, table_hbm, out_hbm, idx_v, rows0, rows1,
                   out_v, sem0, sem1):
    wid = lax.axis_index("s") * 2 + lax.axis_index("c")

    pltpu.sync_copy(idx_hbm.at[pl.ds(wid * PPW, PPW)], idx_v)

    pltpu.async_copy(table_hbm.at[idx_v.at[0]], rows0, sem0)

    def pair_body(i, _):
        p0 = 2 * i
        cp1 = pltpu.async_copy(table_hbm.at[idx_v.at[p0 + 1]], rows1, sem1)
        pltpu.make_async_copy(table_hbm.at[idx_v.at[0]], rows0, sem0).wait()
        _reduce_pair(rows0, out_v, 2 * p0)

        @pl.when(i < PPW // 2 - 1)
        def _():
            pltpu.async_copy(table_hbm.at[idx_v.at[p0 + 2]], rows0, sem0)

        cp1.wait()
        _reduce_pair(rows1, out_v, 2 * p0 + 2)
        return 0

    lax.fori_loop(0, PPW // 2, pair_body, 0)

    pltpu.sync_copy(out_v, out_hbm.at[pl.ds(wid * SPW, SPW)])


def kernel(indices, table):
    idx = jnp.pad(indices.astype(jnp.int32).reshape(NP, 2 * L),
                  ((0, 0), (0, LP - 2 * L)))
    tp = jnp.pad(table, ((0, VP - V), (0, DP - D)))
    return _pooled_lookup(idx, tp)[:, :D]


# speedup vs baseline: 1.0238x; 1.0238x over previous
import functools
import jax, jax.numpy as jnp
import numpy as np
from jax import lax
from jax.experimental import pallas as pl
from jax.experimental.pallas import tpu as pltpu
from jax.experimental.pallas import tpu_sc as plsc

B = 4096
L = 50
NP = B // 2
D = 300
V = 517015
NW = 32
SPW = B // NW
PPW = NP // NW
GP = 304
INV_L = 1.0 / L

_mesh = plsc.VectorSubcoreMesh(core_axis_name="c", subcore_axis_name="s")


def _reduce_pair(rows, out2):
    for half in (0, 1):
        def word_body(j, accs):
            r0 = 3 * j
            a = [accs[u] + rows[r0, pl.ds(u * 16, 16)] for u in range(8)]
            b = [accs[8 + u] + rows[r0 + 1, pl.ds(u * 16, 16)] for u in range(8)]
            c = [accs[16 + u] + rows[r0 + 2, pl.ds(u * 16, 16)] for u in range(3)]
            return tuple(a + b + c)

        init = tuple(jnp.zeros((16,), jnp.float32) for _ in range(19))
        accs = lax.fori_loop(half * L, (half + 1) * L, word_body, init,
                             unroll=5)
        for u in range(8):
            out2[half, pl.ds(u * 16, 16)] = accs[u] * INV_L
        for u in range(8):
            out2[half, pl.ds(128 + u * 16, 16)] = accs[8 + u] * INV_L
        for u in range(3):
            out2[half, pl.ds(256 + u * 16, 16)] = accs[16 + u] * INV_L


@functools.partial(
    pl.kernel,
    out_type=jax.ShapeDtypeStruct((B, 304), jnp.float32),
    mesh=_mesh,
    scratch_types=[
        pltpu.VMEM((PPW, GP), jnp.int32),
        pltpu.VMEM((GP, 128), jnp.float32),
        pltpu.VMEM((GP, 128), jnp.float32),
        pltpu.VMEM((2, 304), jnp.float32),
        pltpu.SemaphoreType.DMA,
        pltpu.SemaphoreType.DMA,
    ],
    compiler_params=pltpu.CompilerParams(use_tc_tiling_on_sc=False),
)
def _pooled_lookup(idx_hbm, s_hbm, out_hbm, idx_v, rows0, rows1,
                   out2, sem0, sem1):
    wid = lax.axis_index("s") * 2 + lax.axis_index("c")
    base = wid * PPW

    pltpu.sync_copy(idx_hbm.at[pl.ds(base, PPW)], idx_v)
    pltpu.async_copy(s_hbm.at[idx_v.at[0]], rows0, sem0)

    def pair_body(i, _):
        p0 = 2 * i
        cp1 = pltpu.async_copy(s_hbm.at[idx_v.at[p0 + 1]], rows1, sem1)
        pltpu.make_async_copy(s_hbm.at[idx_v.at[0]], rows0, sem0).wait()
        _reduce_pair(rows0, out2)
        pltpu.sync_copy(out2, out_hbm.at[pl.ds((base + p0) * 2, 2)])

        @pl.when(i < PPW // 2 - 1)
        def _():
            pltpu.async_copy(s_hbm.at[idx_v.at[p0 + 2]], rows0, sem0)

        cp1.wait()
        _reduce_pair(rows1, out2)
        pltpu.sync_copy(out2, out_hbm.at[pl.ds((base + p0 + 1) * 2, 2)])
        return 0

    lax.fori_loop(0, PPW // 2, pair_body, 0)


def kernel(indices, table):
    idx = indices.astype(jnp.int32).reshape(NP, 2 * L)
    idx3 = (3 * idx[:, :, None]
            + jnp.arange(3, dtype=jnp.int32)[None, None, :]).reshape(NP, 300)
    idx3 = jnp.pad(idx3, ((0, 0), (0, GP - 300)))
    s = jnp.pad(table, ((0, 0), (0, 384 - D))).reshape(3 * V, 128)
    return _pooled_lookup(idx3, s)[:, :D]


# speedup vs baseline: 3.3669x; 3.2887x over previous
import functools

import jax
import jax.numpy as jnp
from jax import lax
from jax.experimental import pallas as pl
from jax.experimental.pallas import tpu as pltpu
from jax.experimental.pallas import tpu_sc as plsc

B = 4096
L = 50
NP = B // 2
D = 300
V = 517015
VB = 253
V2 = VB * 2048
NW = 32
PPW = NP // NW
GP = 304
INV_L = 1.0 / L

_mesh = plsc.VectorSubcoreMesh(core_axis_name="c", subcore_axis_name="s")


def _retile_body(tt_ref, s_ref):
    s_ref[...] = jnp.transpose(tt_ref[...], (1, 0))


@jax.jit
def _retile(tt):
    return pl.pallas_call(
        _retile_body,
        grid=(3, VB),
        in_specs=[pl.BlockSpec((128, 2048), lambda g, j: (g, j))],
        out_specs=pl.BlockSpec((2048, 128), lambda g, j: (g * VB + j, 0)),
        out_shape=jax.ShapeDtypeStruct((3 * V2, 128), jnp.float32),
    )(tt)


def _reduce_pair(rows, out2):
    for half in (0, 1):
        def word_body(j, accs):
            r0 = 3 * j
            a = [accs[u] + rows[r0, pl.ds(u * 16, 16)] for u in range(8)]
            b = [accs[8 + u] + rows[r0 + 1, pl.ds(u * 16, 16)] for u in range(8)]
            c = [accs[16 + u] + rows[r0 + 2, pl.ds(u * 16, 16)] for u in range(3)]
            return tuple(a + b + c)

        init = tuple(jnp.zeros((16,), jnp.float32) for _ in range(19))
        accs = lax.fori_loop(half * L, (half + 1) * L, word_body, init,
                             unroll=5)
        for u in range(8):
            out2[half, pl.ds(u * 16, 16)] = accs[u] * INV_L
        for u in range(8):
            out2[half, pl.ds(128 + u * 16, 16)] = accs[8 + u] * INV_L
        for u in range(3):
            out2[half, pl.ds(256 + u * 16, 16)] = accs[16 + u] * INV_L


@functools.partial(
    pl.kernel,
    out_type=jax.ShapeDtypeStruct((B, 304), jnp.float32),
    mesh=_mesh,
    scratch_types=[
        pltpu.VMEM((PPW, GP), jnp.int32),
        pltpu.VMEM((GP, 128), jnp.float32),
        pltpu.VMEM((GP, 128), jnp.float32),
        pltpu.VMEM((2, 304), jnp.float32),
        pltpu.SemaphoreType.DMA,
        pltpu.SemaphoreType.DMA,
    ],
    compiler_params=pltpu.CompilerParams(use_tc_tiling_on_sc=False),
)
def _pooled_lookup(idx_hbm, s_hbm, out_hbm, idx_v, rows0, rows1,
                   out2, sem0, sem1):
    wid = lax.axis_index("s") * 2 + lax.axis_index("c")
    base = wid * PPW

    pltpu.sync_copy(idx_hbm.at[pl.ds(base, PPW)], idx_v)
    pltpu.async_copy(s_hbm.at[idx_v.at[0]], rows0, sem0)

    def pair_body(i, _):
        p0 = 2 * i
        cp1 = pltpu.async_copy(s_hbm.at[idx_v.at[p0 + 1]], rows1, sem1)
        pltpu.make_async_copy(s_hbm.at[idx_v.at[0]], rows0, sem0).wait()
        _reduce_pair(rows0, out2)
        pltpu.sync_copy(out2, out_hbm.at[pl.ds((base + p0) * 2, 2)])

        @pl.when(i < PPW // 2 - 1)
        def _():
            pltpu.async_copy(s_hbm.at[idx_v.at[p0 + 2]], rows0, sem0)

        cp1.wait()
        _reduce_pair(rows1, out2)
        pltpu.sync_copy(out2, out_hbm.at[pl.ds((base + p0 + 1) * 2, 2)])
        return 0

    lax.fori_loop(0, PPW // 2, pair_body, 0)


def kernel(indices, table):
    idx = indices.astype(jnp.int32).reshape(NP, 2 * L)
    piece = jnp.array([0, V2, 2 * V2], dtype=jnp.int32)
    idx3 = (idx[:, :, None] + piece[None, None, :]).reshape(NP, 300)
    idx3 = jnp.pad(idx3, ((0, 0), (0, GP - 300)))
    s = _retile(table.T)
    return _pooled_lookup(idx3, s)[:, :D]


# speedup vs baseline: 3.3844x; 1.0052x over previous
import functools

import jax
import jax.numpy as jnp
from jax import lax
from jax.experimental import pallas as pl
from jax.experimental.pallas import tpu as pltpu
from jax.experimental.pallas import tpu_sc as plsc

B = 4096
L = 50
D = 300
V = 517015
VB = 253
V2 = VB * 2048
NW = 32
SPW = B // NW
GP = 152
INV_L = 1.0 / L

_mesh = plsc.VectorSubcoreMesh(core_axis_name="c", subcore_axis_name="s")


def _retile_body(tt_ref, s_ref):
    s_ref[...] = jnp.transpose(tt_ref[...], (1, 0))


@jax.jit
def _retile(tt):
    return pl.pallas_call(
        _retile_body,
        grid=(3, VB),
        in_specs=[pl.BlockSpec((128, 2048), lambda g, j: (g, j))],
        out_specs=pl.BlockSpec((2048, 128), lambda g, j: (g * VB + j, 0)),
        out_shape=jax.ShapeDtypeStruct((3 * V2, 128), jnp.float32),
    )(tt)


def _reduce_sent(rows, out_v, w):
    def word_body(j, accs):
        r0 = 3 * j
        a = [accs[u] + rows[r0, pl.ds(u * 16, 16)] for u in range(8)]
        b = [accs[8 + u] + rows[r0 + 1, pl.ds(u * 16, 16)] for u in range(8)]
        c = [accs[16 + u] + rows[r0 + 2, pl.ds(u * 16, 16)] for u in range(3)]
        return tuple(a + b + c)

    init = tuple(jnp.zeros((16,), jnp.float32) for _ in range(19))
    accs = lax.fori_loop(0, L, word_body, init, unroll=5)
    for u in range(8):
        out_v[w, pl.ds(u * 16, 16)] = accs[u] * INV_L
    for u in range(8):
        out_v[w, pl.ds(128 + u * 16, 16)] = accs[8 + u] * INV_L
    for u in range(3):
        out_v[w, pl.ds(256 + u * 16, 16)] = accs[16 + u] * INV_L


@functools.partial(
    pl.kernel,
    out_type=jax.ShapeDtypeStruct((B, 304), jnp.float32),
    mesh=_mesh,
    scratch_types=[
        pltpu.VMEM((SPW, GP), jnp.int32),
        pltpu.VMEM((GP, 128), jnp.float32),
        pltpu.VMEM((GP, 128), jnp.float32),
        pltpu.VMEM((SPW, 304), jnp.float32),
        pltpu.SemaphoreType.DMA,
        pltpu.SemaphoreType.DMA,
    ],
    compiler_params=pltpu.CompilerParams(use_tc_tiling_on_sc=False),
)
def _pooled_lookup(idx_hbm, s_hbm, out_hbm, idx_v, rows0, rows1,
                   out_v, sem0, sem1):
    wid = lax.axis_index("s") * 2 + lax.axis_index("c")
    base = wid * SPW

    pltpu.sync_copy(idx_hbm.at[pl.ds(base, SPW)], idx_v)
    pltpu.async_copy(s_hbm.at[idx_v.at[0]], rows0, sem0)

    def sent_body(i, _):
        w0 = 2 * i
        cp1 = pltpu.async_copy(s_hbm.at[idx_v.at[w0 + 1]], rows1, sem1)
        pltpu.make_async_copy(s_hbm.at[idx_v.at[0]], rows0, sem0).wait()
        _reduce_sent(rows0, out_v, w0)

        @pl.when(i < SPW // 2 - 1)
        def _():
            pltpu.async_copy(s_hbm.at[idx_v.at[w0 + 2]], rows0, sem0)

        cp1.wait()
        _reduce_sent(rows1, out_v, w0 + 1)
        return 0

    lax.fori_loop(0, SPW // 2, sent_body, 0)

    pltpu.sync_copy(out_v, out_hbm.at[pl.ds(base, SPW)])


def kernel(indices, table):
    idx = indices.astype(jnp.int32)
    piece = jnp.array([0, V2, 2 * V2], dtype=jnp.int32)
    idx3 = (idx[:, :, None] + piece[None, None, :]).reshape(B, 3 * L)
    idx3 = jnp.pad(idx3, ((0, 0), (0, GP - 3 * L)))
    s = _retile(table.T)
    return _pooled_lookup(idx3, s)[:, :D]
